# baseline (speedup 1.0000x reference)
import functools

import jax
import jax.numpy as jnp
from jax import lax
from jax.experimental import pallas as pl
from jax.experimental.pallas import tpu as pltpu
from jax.experimental.pallas import tpu_sc as plsc

N = 10000
E = 160000
T_NODE = 128
T_OP = 16
H = 32
OUT = 16
L = 3

GW = 128
BE = 1600
NB = E // BE
LN_EPS = 1e-5

NC = 2
NS = 16
NW = NC * NS
CHUNK = 128
ROWS = E // CHUNK
IDXR = 1280
RW = IDXR // NW
K_SUP = 13
NPR = N // NS

def _worker_base(c, s):
    wid = s * NC + c
    return wid, wid * 39 + jnp.minimum(wid, 2)


@functools.lru_cache(maxsize=None)
def _build_gather_sc(width):
    if width >= GW:
        plan = (6, 6, 6, 6, 6, 6, 4)
        tc_tiled = True
        n_out = IDXR * CHUNK
    else:
        plan = (K_SUP, K_SUP, K_SUP)
        tc_tiled = False
        n_out = E
    bufk = max(plan)

    @functools.partial(
        pl.kernel,
        out_type=jax.ShapeDtypeStruct((n_out, width), jnp.float32),
        mesh=plsc.VectorSubcoreMesh(core_axis_name="c", subcore_axis_name="s",
                                    num_cores=NC, num_subcores=NS),
        scratch_types=[
            pltpu.VMEM((RW, CHUNK), jnp.int32),
            pltpu.VMEM((bufk * CHUNK, width), jnp.float32),
            pltpu.SemaphoreType.DMA,
        ],
        compiler_params=pltpu.CompilerParams(use_tc_tiling_on_sc=tc_tiled),
    )
    def _gather_sc(table_hbm, src_hbm, out_hbm, idx_v, buf_v, sem):
        c = lax.axis_index("c")
        s = lax.axis_index("s")
        if tc_tiled:
            wid = s * NC + c
            row_base = wid * RW
        else:
            wid, row_base = _worker_base(c, s)
        pltpu.sync_copy(src_hbm.at[pl.ds(row_base, RW)], idx_v)

        off = 0
        for k_sup in plan:
            cds = []
            for k in range(k_sup):
                cds.append(pltpu.async_copy(
                    table_hbm.at[idx_v.at[off + k]],
                    buf_v.at[pl.ds(k * CHUNK, CHUNK)], sem))
            for cd in cds:
                cd.wait()
            pltpu.sync_copy(
                buf_v.at[pl.ds(0, k_sup * CHUNK)],
                out_hbm.at[pl.ds((row_base + off) * CHUNK, k_sup * CHUNK)])
            off += k_sup

        if not tc_tiled:
            @pl.when(wid < 2)
            def _tail():
                pltpu.async_copy(table_hbm.at[idx_v.at[39]],
                                 buf_v.at[pl.ds(0, CHUNK)], sem).wait()
                pltpu.sync_copy(
                    buf_v.at[pl.ds(0, CHUNK)],
                    out_hbm.at[pl.ds((row_base + 39) * CHUNK, CHUNK)])

    return _gather_sc


@functools.lru_cache(maxsize=None)
def _build_scatter_sc():
    @functools.partial(
        pl.kernel,
        out_type=jax.ShapeDtypeStruct((NC, N, 2 * OUT), jnp.float32),
        mesh=plsc.VectorSubcoreMesh(core_axis_name="c", subcore_axis_name="s",
                                    num_cores=NC, num_subcores=NS),
        scratch_types=[
            pltpu.VMEM_SHARED((N, 2 * OUT), jnp.float32),
            pltpu.VMEM((RW, CHUNK), jnp.int32),
            pltpu.VMEM((K_SUP * CHUNK, 2 * OUT), jnp.float32),
            pltpu.SemaphoreType.DMA,
        ],
        compiler_params=pltpu.CompilerParams(use_tc_tiling_on_sc=False),
    )
    def _scatter_sc(m_hbm, dst_hbm, zeros_hbm, out_hbm, acc_sh, idx_v,
                    buf_v, sem):
        c = lax.axis_index("c")
        s = lax.axis_index("s")
        wid, row_base = _worker_base(c, s)

        @pl.when(s == 0)
        def _init():
            pltpu.sync_copy(zeros_hbm, acc_sh)

        pltpu.sync_copy(dst_hbm.at[pl.ds(row_base, RW)], idx_v)
        plsc.subcore_barrier()

        for sup in range(3):
            cds = []
            for k in range(K_SUP):
                cds.append(pltpu.async_copy(
                    m_hbm.at[pl.ds((row_base + sup * K_SUP + k) * CHUNK,
                                   CHUNK)],
                    buf_v.at[pl.ds(k * CHUNK, CHUNK)], sem))
            for cd in cds:
                cd.wait()
            for k in range(K_SUP):
                pltpu.sync_copy(buf_v.at[pl.ds(k * CHUNK, CHUNK)],
                                acc_sh.at[idx_v.at[sup * K_SUP + k]], add=True)

        @pl.when(wid < 2)
        def _tail():
            pltpu.async_copy(
                m_hbm.at[pl.ds((row_base + 39) * CHUNK, CHUNK)],
                buf_v.at[pl.ds(0, CHUNK)], sem).wait()
            pltpu.sync_copy(buf_v.at[pl.ds(0, CHUNK)],
                            acc_sh.at[idx_v.at[39]], add=True)

        plsc.subcore_barrier()
        pltpu.sync_copy(acc_sh.at[pl.ds(s * NPR, NPR)],
                        out_hbm.at[c, pl.ds(s * NPR, NPR)])

    return _scatter_sc


def _ln_relu_cat(h, g2, b2):
    mu = jnp.mean(h, axis=-1, keepdims=True)
    d = h - mu
    var = jnp.mean(d * d, axis=-1, keepdims=True)
    hn = d * lax.rsqrt(var + LN_EPS)
    h2n = jax.nn.relu(hn * g2[0:1, :] + b2[0:1, :])
    h2r = jax.nn.relu(hn * g2[1:2, :] + b2[1:2, :])
    pad = jnp.zeros((h.shape[0], GW - 2 * H), jnp.float32)
    return jnp.concatenate([h2n, h2r, pad], axis=1)


def _dense_first_body(x_ref, w0_ref, g_ref, b_ref, rt_ref, bi_ref,
                      h2cat_ref, hbase_ref):
    h = jnp.dot(x_ref[...], w0_ref[...], preferred_element_type=jnp.float32)
    h2cat = _ln_relu_cat(h, g_ref[...], b_ref[...])
    h2cat_ref[...] = h2cat
    hbase_ref[...] = h + jnp.dot(h2cat, rt_ref[...],
                                 preferred_element_type=jnp.float32) + bi_ref[...]


def _dense_mid_body(hb_ref, parts_ref, g_ref, b_ref, rt_ref, bi_ref,
                    h2cat_ref, hbase_ref):
    h = hb_ref[...] + parts_ref[0] + parts_ref[1]
    h2cat = _ln_relu_cat(h, g_ref[...], b_ref[...])
    h2cat_ref[...] = h2cat
    hbase_ref[...] = h + jnp.dot(h2cat, rt_ref[...],
                                 preferred_element_type=jnp.float32) + bi_ref[...]


def _final_body(hb_ref, parts_ref, out_ref):
    out_ref[...] = hb_ref[...] + parts_ref[0] + parts_ref[1]


def _edge_body(xj_ref, ea_ref, ear_ref, w1_ref, r2_ref, s2_ref, eb_ref, m_ref):
    xj = xj_ref[...]
    ea_cat = jnp.concatenate([ea_ref[...], ear_ref[...]], axis=1)
    tmp = jnp.dot(xj, w1_ref[...], preferred_element_type=jnp.float32)
    arep = jnp.dot(ea_cat, r2_ref[...], preferred_element_type=jnp.float32)
    p = tmp * arep
    m_ref[...] = (jnp.dot(p, s2_ref[...], preferred_element_type=jnp.float32)
                  + jnp.dot(xj, eb_ref[...], preferred_element_type=jnp.float32))


def _dense_first(x, w0, g2, b2, rtcat, bicat, interpret=False):
    return pl.pallas_call(
        _dense_first_body,
        out_shape=(jax.ShapeDtypeStruct((N, GW), jnp.float32),
                   jax.ShapeDtypeStruct((N, H), jnp.float32)),
        interpret=interpret,
    )(x, w0, g2, b2, rtcat, bicat)


def _dense_mid(hb, parts, g2, b2, rtcat, bicat, interpret=False):
    return pl.pallas_call(
        _dense_mid_body,
        out_shape=(jax.ShapeDtypeStruct((N, GW), jnp.float32),
                   jax.ShapeDtypeStruct((N, H), jnp.float32)),
        interpret=interpret,
    )(hb, parts, g2, b2, rtcat, bicat)


def _final(hb, parts, interpret=False):
    return pl.pallas_call(
        _final_body,
        out_shape=jax.ShapeDtypeStruct((N, H), jnp.float32),
        interpret=interpret,
    )(hb, parts)


def _edge_compute(xj, ea, ea_rev, w1cat, r2, s2cat, ebcat, interpret=False):
    return pl.pallas_call(
        _edge_body,
        grid=(NB,),
        in_specs=[
            pl.BlockSpec((BE, GW), lambda i: (i, 0)),
            pl.BlockSpec((BE, T_OP), lambda i: (i, 0)),
            pl.BlockSpec((BE, T_OP), lambda i: (i, 0)),
            pl.BlockSpec((GW, 2 * T_OP * OUT), lambda i: (0, 0)),
            pl.BlockSpec((2 * T_OP, 2 * T_OP * OUT), lambda i: (0, 0)),
            pl.BlockSpec((2 * T_OP * OUT, 2 * OUT), lambda i: (0, 0)),
            pl.BlockSpec((GW, 2 * OUT), lambda i: (0, 0)),
        ],
        out_specs=pl.BlockSpec((BE, 2 * OUT), lambda i: (i, 0)),
        out_shape=jax.ShapeDtypeStruct((E, 2 * OUT), jnp.float32),
        interpret=interpret,
    )(xj, ea, ea_rev, w1cat, r2, s2cat, ebcat)


def _gather_nodes(h2cat, src2d):
    return _build_gather_sc(GW)(h2cat, src2d)


def _scatter_messages(m, dst2d, zeros_acc):
    return _build_scatter_sc()(m, dst2d, zeros_acc)


def _prep_weights(edge_W, edge_b, root, bias_p, ln_gamma, ln_beta):
    def w1(eW):
        return eW.reshape(T_OP, H, OUT).transpose(1, 2, 0).reshape(H, OUT * T_OP)

    eye_t = jnp.eye(T_OP, dtype=jnp.float32)
    r_b = jnp.tile(eye_t, (1, OUT))
    s_b = jnp.kron(jnp.eye(OUT, dtype=jnp.float32),
                   jnp.ones((T_OP, 1), jnp.float32))
    z = jnp.zeros((T_OP, OUT * T_OP), jnp.float32)
    r2 = jnp.block([[r_b, z], [z, r_b]])
    zs = jnp.zeros((OUT * T_OP, OUT), jnp.float32)
    s2 = jnp.block([[s_b, zs], [zs, s_b]])

    kpad = ((0, GW - 2 * H), (0, 0))
    per_layer = []
    for l in range(L):
        w1n, w1r = w1(edge_W[l, 0]), w1(edge_W[l, 1])
        zw = jnp.zeros((H, OUT * T_OP), jnp.float32)
        w1cat = jnp.pad(jnp.block([[w1n, zw], [zw, w1r]]), kpad)
        ebn = edge_b[l, 0].reshape(H, OUT)
        ebr = edge_b[l, 1].reshape(H, OUT)
        zb = jnp.zeros((H, OUT), jnp.float32)
        ebcat = jnp.pad(jnp.block([[ebn, zb], [zb, ebr]]), kpad)
        zr = jnp.zeros((H, OUT), jnp.float32)
        rtcat = jnp.pad(jnp.block([[root[l, 0], zr], [zr, root[l, 1]]]),
                        kpad)
        bicat = jnp.concatenate([bias_p[l, 0], bias_p[l, 1]]).reshape(1, 2 * OUT)
        g2 = ln_gamma[l]
        b2 = ln_beta[l]
        per_layer.append((w1cat, ebcat, rtcat, bicat, g2, b2))
    return r2, s2, per_layer


def _encoder(x, edge_index, edge_attr, W0, ln_gamma, ln_beta, edge_W, edge_b,
             root, bias_p, interpret=False):
    r2, s2, per_layer = _prep_weights(edge_W, edge_b, root, bias_p,
                                      ln_gamma, ln_beta)

    src2d = jnp.pad(edge_index[0].reshape(ROWS, CHUNK),
                    ((0, IDXR - ROWS), (0, 0)))
    dst2d = jnp.pad(edge_index[1].reshape(ROWS, CHUNK),
                    ((0, IDXR - ROWS), (0, 0)))
    zeros_acc = jnp.zeros((N, 2 * OUT), jnp.float32)
    rev2d = jnp.pad(
        ((E - 1) - jnp.arange(E, dtype=jnp.int32)).reshape(ROWS, CHUNK),
        ((0, IDXR - ROWS), (0, 0)))
    if interpret:
        ea_rev = edge_attr[::-1]
    else:
        ea_rev = _build_gather_sc(T_OP)(edge_attr, rev2d)

    hb = None
    parts = None
    for l in range(L):
        w1cat, ebcat, rtcat, bicat, g2, b2 = per_layer[l]
        if l == 0:
            h2cat, hb = _dense_first(x, W0, g2, b2, rtcat, bicat,
                                     interpret=interpret)
        else:
            h2cat, hb = _dense_mid(hb, parts, g2, b2, rtcat, bicat,
                                   interpret=interpret)
        if interpret:
            xj = jnp.take(h2cat, src2d[:ROWS].reshape(-1), axis=0)
        else:
            xj = _gather_nodes(h2cat, src2d)
        m = _edge_compute(xj, edge_attr, ea_rev, w1cat, r2, s2, ebcat,
                          interpret=interpret)
        if interpret:
            aggr = jnp.zeros((N, 2 * OUT), jnp.float32).at[
                dst2d[:ROWS].reshape(-1)].add(m)
            parts = jnp.stack([aggr, jnp.zeros_like(aggr)])
        else:
            parts = _scatter_messages(m, dst2d, zeros_acc)
    return _final(hb, parts, interpret=interpret)


def kernel(x, edge_index, edge_attr, W0, ln_gamma, ln_beta, edge_W, edge_b,
           root, bias_p):
    return _encoder(x, edge_index, edge_attr, W0, ln_gamma, ln_beta,
                    edge_W, edge_b, root, bias_p)


# speedup vs baseline: 1.2654x; 1.2654x over previous
import functools

import jax
import jax.numpy as jnp
from jax import lax
from jax.experimental import pallas as pl
from jax.experimental.pallas import tpu as pltpu
from jax.experimental.pallas import tpu_sc as plsc

N = 10000
E = 160000
T_NODE = 128
T_OP = 16
H = 32
OUT = 16
L = 3

BE = 1600
NB = E // BE
LN_EPS = 1e-5

NC = 2
NS = 16
NW = NC * NS
CHUNK = 128
ROWS = E // CHUNK
IDXR = 1280
RW = IDXR // NW
K_SUP = 13
NPR = N // NS

def _worker_base(c, s):
    wid = s * NC + c
    return wid, wid * 39 + jnp.minimum(wid, 2)


@functools.lru_cache(maxsize=None)
def _build_gather_sc(width):
    @functools.partial(
        pl.kernel,
        out_type=jax.ShapeDtypeStruct((E, width), jnp.float32),
        mesh=plsc.VectorSubcoreMesh(core_axis_name="c", subcore_axis_name="s",
                                    num_cores=NC, num_subcores=NS),
        scratch_types=[
            pltpu.VMEM((RW, CHUNK), jnp.int32),
            pltpu.VMEM((K_SUP * CHUNK, width), jnp.float32),
            pltpu.VMEM((CHUNK, width), jnp.float32),
            pltpu.SemaphoreType.DMA,
        ],
        compiler_params=pltpu.CompilerParams(use_tc_tiling_on_sc=False),
    )
    def _gather_sc(table_hbm, src_hbm, out_hbm, idx_v, buf_v, tail_v, sem):
        c = lax.axis_index("c")
        s = lax.axis_index("s")
        wid, row_base = _worker_base(c, s)
        pltpu.sync_copy(src_hbm.at[pl.ds(row_base, RW)], idx_v)

        for sup in range(3):
            cds = []
            for k in range(K_SUP):
                cds.append(pltpu.async_copy(
                    table_hbm.at[idx_v.at[sup * K_SUP + k]],
                    buf_v.at[pl.ds(k * CHUNK, CHUNK)], sem))
            for cd in cds:
                cd.wait()
            pltpu.sync_copy(
                buf_v,
                out_hbm.at[pl.ds((row_base + sup * K_SUP) * CHUNK,
                                 K_SUP * CHUNK)])

        @pl.when(wid < 2)
        def _tail():
            pltpu.async_copy(table_hbm.at[idx_v.at[39]], tail_v, sem).wait()
            pltpu.sync_copy(tail_v,
                            out_hbm.at[pl.ds((row_base + 39) * CHUNK, CHUNK)])

    return _gather_sc


@functools.lru_cache(maxsize=None)
def _build_scatter_sc():
    @functools.partial(
        pl.kernel,
        out_type=jax.ShapeDtypeStruct((NC, N, 2 * OUT), jnp.float32),
        mesh=plsc.VectorSubcoreMesh(core_axis_name="c", subcore_axis_name="s",
                                    num_cores=NC, num_subcores=NS),
        scratch_types=[
            pltpu.VMEM_SHARED((N, 2 * OUT), jnp.float32),
            pltpu.VMEM((RW, CHUNK), jnp.int32),
            pltpu.VMEM((K_SUP * CHUNK, 2 * OUT), jnp.float32),
            pltpu.SemaphoreType.DMA,
        ],
        compiler_params=pltpu.CompilerParams(use_tc_tiling_on_sc=False),
    )
    def _scatter_sc(m_hbm, dst_hbm, zeros_hbm, out_hbm, acc_sh, idx_v,
                    buf_v, sem):
        c = lax.axis_index("c")
        s = lax.axis_index("s")
        wid, row_base = _worker_base(c, s)

        @pl.when(s == 0)
        def _init():
            pltpu.sync_copy(zeros_hbm, acc_sh)

        pltpu.sync_copy(dst_hbm.at[pl.ds(row_base, RW)], idx_v)
        plsc.subcore_barrier()

        for sup in range(3):
            cds = []
            for k in range(K_SUP):
                cds.append(pltpu.async_copy(
                    m_hbm.at[pl.ds((row_base + sup * K_SUP + k) * CHUNK,
                                   CHUNK)],
                    buf_v.at[pl.ds(k * CHUNK, CHUNK)], sem))
            for cd in cds:
                cd.wait()
            for k in range(K_SUP):
                pltpu.sync_copy(buf_v.at[pl.ds(k * CHUNK, CHUNK)],
                                acc_sh.at[idx_v.at[sup * K_SUP + k]], add=True)

        @pl.when(wid < 2)
        def _tail():
            pltpu.async_copy(
                m_hbm.at[pl.ds((row_base + 39) * CHUNK, CHUNK)],
                buf_v.at[pl.ds(0, CHUNK)], sem).wait()
            pltpu.sync_copy(buf_v.at[pl.ds(0, CHUNK)],
                            acc_sh.at[idx_v.at[39]], add=True)

        plsc.subcore_barrier()
        pltpu.sync_copy(acc_sh.at[pl.ds(s * NPR, NPR)],
                        out_hbm.at[c, pl.ds(s * NPR, NPR)])

    return _scatter_sc


def _ln_relu_cat(h, g2, b2):
    mu = jnp.mean(h, axis=-1, keepdims=True)
    d = h - mu
    var = jnp.mean(d * d, axis=-1, keepdims=True)
    hn = d * lax.rsqrt(var + LN_EPS)
    h2n = jax.nn.relu(hn * g2[0:1, :] + b2[0:1, :])
    h2r = jax.nn.relu(hn * g2[1:2, :] + b2[1:2, :])
    return jnp.concatenate([h2n, h2r], axis=1)


def _dense_first_body(x_ref, w0_ref, g_ref, b_ref, rt_ref, bi_ref,
                      h2cat_ref, hbase_ref):
    h = jnp.dot(x_ref[...], w0_ref[...], preferred_element_type=jnp.float32)
    h2cat = _ln_relu_cat(h, g_ref[...], b_ref[...])
    h2cat_ref[...] = h2cat
    hbase_ref[...] = h + jnp.dot(h2cat, rt_ref[...],
                                 preferred_element_type=jnp.float32) + bi_ref[...]


def _dense_mid_body(hb_ref, parts_ref, g_ref, b_ref, rt_ref, bi_ref,
                    h2cat_ref, hbase_ref):
    h = hb_ref[...] + parts_ref[0] + parts_ref[1]
    h2cat = _ln_relu_cat(h, g_ref[...], b_ref[...])
    h2cat_ref[...] = h2cat
    hbase_ref[...] = h + jnp.dot(h2cat, rt_ref[...],
                                 preferred_element_type=jnp.float32) + bi_ref[...]


def _final_body(hb_ref, parts_ref, out_ref):
    out_ref[...] = hb_ref[...] + parts_ref[0] + parts_ref[1]


def _edge_body(xj_ref, ea_ref, ear_ref, w1_ref, r2_ref, s2_ref, eb_ref, m_ref):
    xj = xj_ref[...]
    ea_cat = jnp.concatenate([ea_ref[...], ear_ref[...]], axis=1)
    tmp = jnp.dot(xj, w1_ref[...], preferred_element_type=jnp.float32)
    arep = jnp.dot(ea_cat, r2_ref[...], preferred_element_type=jnp.float32)
    p = tmp * arep
    m_ref[...] = (jnp.dot(p, s2_ref[...], preferred_element_type=jnp.float32)
                  + jnp.dot(xj, eb_ref[...], preferred_element_type=jnp.float32))


def _dense_first(x, w0, g2, b2, rtcat, bicat, interpret=False):
    return pl.pallas_call(
        _dense_first_body,
        out_shape=(jax.ShapeDtypeStruct((N, 2 * H), jnp.float32),
                   jax.ShapeDtypeStruct((N, H), jnp.float32)),
        interpret=interpret,
    )(x, w0, g2, b2, rtcat, bicat)


def _dense_mid(hb, parts, g2, b2, rtcat, bicat, interpret=False):
    return pl.pallas_call(
        _dense_mid_body,
        out_shape=(jax.ShapeDtypeStruct((N, 2 * H), jnp.float32),
                   jax.ShapeDtypeStruct((N, H), jnp.float32)),
        interpret=interpret,
    )(hb, parts, g2, b2, rtcat, bicat)


def _final(hb, parts, interpret=False):
    return pl.pallas_call(
        _final_body,
        out_shape=jax.ShapeDtypeStruct((N, H), jnp.float32),
        interpret=interpret,
    )(hb, parts)


def _edge_compute(xj, ea, ea_rev, w1cat, r2, s2cat, ebcat, interpret=False):
    return pl.pallas_call(
        _edge_body,
        grid=(NB,),
        in_specs=[
            pl.BlockSpec((BE, 2 * H), lambda i: (i, 0)),
            pl.BlockSpec((BE, T_OP), lambda i: (i, 0)),
            pl.BlockSpec((BE, T_OP), lambda i: (i, 0)),
            pl.BlockSpec((2 * H, 2 * T_OP * OUT), lambda i: (0, 0)),
            pl.BlockSpec((2 * T_OP, 2 * T_OP * OUT), lambda i: (0, 0)),
            pl.BlockSpec((2 * T_OP * OUT, 2 * OUT), lambda i: (0, 0)),
            pl.BlockSpec((2 * H, 2 * OUT), lambda i: (0, 0)),
        ],
        out_specs=pl.BlockSpec((BE, 2 * OUT), lambda i: (i, 0)),
        out_shape=jax.ShapeDtypeStruct((E, 2 * OUT), jnp.float32),
        interpret=interpret,
    )(xj, ea, ea_rev, w1cat, r2, s2cat, ebcat)


def _gather_nodes(h2cat, src2d):
    return _build_gather_sc(2 * H)(h2cat, src2d)


def _scatter_messages(m, dst2d, zeros_acc):
    return _build_scatter_sc()(m, dst2d, zeros_acc)


def _prep_weights(edge_W, edge_b, root, bias_p, ln_gamma, ln_beta):
    def w1(eW):
        return eW.reshape(T_OP, H, OUT).transpose(1, 2, 0).reshape(H, OUT * T_OP)

    eye_t = jnp.eye(T_OP, dtype=jnp.float32)
    r_b = jnp.tile(eye_t, (1, OUT))
    s_b = jnp.kron(jnp.eye(OUT, dtype=jnp.float32),
                   jnp.ones((T_OP, 1), jnp.float32))
    z = jnp.zeros((T_OP, OUT * T_OP), jnp.float32)
    r2 = jnp.block([[r_b, z], [z, r_b]])
    zs = jnp.zeros((OUT * T_OP, OUT), jnp.float32)
    s2 = jnp.block([[s_b, zs], [zs, s_b]])

    per_layer = []
    for l in range(L):
        w1n, w1r = w1(edge_W[l, 0]), w1(edge_W[l, 1])
        zw = jnp.zeros((H, OUT * T_OP), jnp.float32)
        w1cat = jnp.block([[w1n, zw], [zw, w1r]])
        ebn = edge_b[l, 0].reshape(H, OUT)
        ebr = edge_b[l, 1].reshape(H, OUT)
        zb = jnp.zeros((H, OUT), jnp.float32)
        ebcat = jnp.block([[ebn, zb], [zb, ebr]])
        zr = jnp.zeros((H, OUT), jnp.float32)
        rtcat = jnp.block([[root[l, 0], zr], [zr, root[l, 1]]])
        bicat = jnp.concatenate([bias_p[l, 0], bias_p[l, 1]]).reshape(1, 2 * OUT)
        g2 = ln_gamma[l]
        b2 = ln_beta[l]
        per_layer.append((w1cat, ebcat, rtcat, bicat, g2, b2))
    return r2, s2, per_layer


def _encoder(x, edge_index, edge_attr, W0, ln_gamma, ln_beta, edge_W, edge_b,
             root, bias_p, interpret=False):
    r2, s2, per_layer = _prep_weights(edge_W, edge_b, root, bias_p,
                                      ln_gamma, ln_beta)

    src2d = jnp.pad(edge_index[0].reshape(ROWS, CHUNK),
                    ((0, IDXR - ROWS), (0, 0)))
    dst2d = jnp.pad(edge_index[1].reshape(ROWS, CHUNK),
                    ((0, IDXR - ROWS), (0, 0)))
    zeros_acc = jnp.zeros((N, 2 * OUT), jnp.float32)
    rev2d = jnp.pad(
        ((E - 1) - jnp.arange(E, dtype=jnp.int32)).reshape(ROWS, CHUNK),
        ((0, IDXR - ROWS), (0, 0)))
    if interpret:
        ea_rev = edge_attr[::-1]
    else:
        ea_rev = _build_gather_sc(T_OP)(edge_attr, rev2d)

    hb = None
    parts = None
    for l in range(L):
        w1cat, ebcat, rtcat, bicat, g2, b2 = per_layer[l]
        if l == 0:
            h2cat, hb = _dense_first(x, W0, g2, b2, rtcat, bicat,
                                     interpret=interpret)
        else:
            h2cat, hb = _dense_mid(hb, parts, g2, b2, rtcat, bicat,
                                   interpret=interpret)
        if interpret:
            xj = jnp.take(h2cat, src2d[:ROWS].reshape(-1), axis=0)
        else:
            xj = _gather_nodes(h2cat, src2d)
        m = _edge_compute(xj, edge_attr, ea_rev, w1cat, r2, s2, ebcat,
                          interpret=interpret)
        if interpret:
            aggr = jnp.zeros((N, 2 * OUT), jnp.float32).at[
                dst2d[:ROWS].reshape(-1)].add(m)
            parts = jnp.stack([aggr, jnp.zeros_like(aggr)])
        else:
            parts = _scatter_messages(m, dst2d, zeros_acc)
    return _final(hb, parts, interpret=interpret)


def kernel(x, edge_index, edge_attr, W0, ln_gamma, ln_beta, edge_W, edge_b,
           root, bias_p):
    return _encoder(x, edge_index, edge_attr, W0, ln_gamma, ln_beta,
                    edge_W, edge_b, root, bias_p)


# speedup vs baseline: 1.3507x; 1.0674x over previous
import functools

import jax
import jax.numpy as jnp
from jax import lax
from jax.experimental import pallas as pl
from jax.experimental.pallas import tpu as pltpu
from jax.experimental.pallas import tpu_sc as plsc

N = 10000
E = 160000
T_NODE = 128
T_OP = 16
H = 32
OUT = 16
L = 3

BE = 3200
NB = E // BE
LN_EPS = 1e-5

NC = 2
NS = 16
NW = NC * NS
CHUNK = 128
ROWS = E // CHUNK
IDXR = 1280
RW = IDXR // NW
K_SUP = 13
NPR = N // NS

def _worker_base(c, s):
    wid = s * NC + c
    return wid, wid * 39 + jnp.minimum(wid, 2)


@functools.lru_cache(maxsize=None)
def _build_gather_sc(width):
    @functools.partial(
        pl.kernel,
        out_type=jax.ShapeDtypeStruct((E, width), jnp.float32),
        mesh=plsc.VectorSubcoreMesh(core_axis_name="c", subcore_axis_name="s",
                                    num_cores=NC, num_subcores=NS),
        scratch_types=[
            pltpu.VMEM((RW, CHUNK), jnp.int32),
            pltpu.VMEM((K_SUP * CHUNK, width), jnp.float32),
            pltpu.VMEM((CHUNK, width), jnp.float32),
            pltpu.SemaphoreType.DMA,
        ],
        compiler_params=pltpu.CompilerParams(use_tc_tiling_on_sc=False),
    )
    def _gather_sc(table_hbm, src_hbm, out_hbm, idx_v, buf_v, tail_v, sem):
        c = lax.axis_index("c")
        s = lax.axis_index("s")
        wid, row_base = _worker_base(c, s)
        pltpu.sync_copy(src_hbm.at[pl.ds(row_base, RW)], idx_v)

        for sup in range(3):
            cds = []
            for k in range(K_SUP):
                cds.append(pltpu.async_copy(
                    table_hbm.at[idx_v.at[sup * K_SUP + k]],
                    buf_v.at[pl.ds(k * CHUNK, CHUNK)], sem))
            for cd in cds:
                cd.wait()
            pltpu.sync_copy(
                buf_v,
                out_hbm.at[pl.ds((row_base + sup * K_SUP) * CHUNK,
                                 K_SUP * CHUNK)])

        @pl.when(wid < 2)
        def _tail():
            pltpu.async_copy(table_hbm.at[idx_v.at[39]], tail_v, sem).wait()
            pltpu.sync_copy(tail_v,
                            out_hbm.at[pl.ds((row_base + 39) * CHUNK, CHUNK)])

    return _gather_sc


@functools.lru_cache(maxsize=None)
def _build_scatter_sc():
    @functools.partial(
        pl.kernel,
        out_type=jax.ShapeDtypeStruct((NC, N, 2 * OUT), jnp.float32),
        mesh=plsc.VectorSubcoreMesh(core_axis_name="c", subcore_axis_name="s",
                                    num_cores=NC, num_subcores=NS),
        scratch_types=[
            pltpu.VMEM_SHARED((N, 2 * OUT), jnp.float32),
            pltpu.VMEM((RW, CHUNK), jnp.int32),
            pltpu.VMEM((K_SUP * CHUNK, 2 * OUT), jnp.float32),
            pltpu.SemaphoreType.DMA,
        ],
        compiler_params=pltpu.CompilerParams(use_tc_tiling_on_sc=False),
    )
    def _scatter_sc(m_hbm, dst_hbm, zeros_hbm, out_hbm, acc_sh, idx_v,
                    buf_v, sem):
        c = lax.axis_index("c")
        s = lax.axis_index("s")
        wid, row_base = _worker_base(c, s)

        @pl.when(s == 0)
        def _init():
            pltpu.sync_copy(zeros_hbm, acc_sh)

        pltpu.sync_copy(dst_hbm.at[pl.ds(row_base, RW)], idx_v)
        plsc.subcore_barrier()

        for sup in range(3):
            cds = []
            for k in range(K_SUP):
                cds.append(pltpu.async_copy(
                    m_hbm.at[pl.ds((row_base + sup * K_SUP + k) * CHUNK,
                                   CHUNK)],
                    buf_v.at[pl.ds(k * CHUNK, CHUNK)], sem))
            for cd in cds:
                cd.wait()
            for k in range(K_SUP):
                pltpu.sync_copy(buf_v.at[pl.ds(k * CHUNK, CHUNK)],
                                acc_sh.at[idx_v.at[sup * K_SUP + k]], add=True)

        @pl.when(wid < 2)
        def _tail():
            pltpu.async_copy(
                m_hbm.at[pl.ds((row_base + 39) * CHUNK, CHUNK)],
                buf_v.at[pl.ds(0, CHUNK)], sem).wait()
            pltpu.sync_copy(buf_v.at[pl.ds(0, CHUNK)],
                            acc_sh.at[idx_v.at[39]], add=True)

        plsc.subcore_barrier()
        pltpu.sync_copy(acc_sh.at[pl.ds(s * NPR, NPR)],
                        out_hbm.at[c, pl.ds(s * NPR, NPR)])

    return _scatter_sc


def _ln_relu_cat(h, g2, b2):
    mu = jnp.mean(h, axis=-1, keepdims=True)
    d = h - mu
    var = jnp.mean(d * d, axis=-1, keepdims=True)
    hn = d * lax.rsqrt(var + LN_EPS)
    h2n = jax.nn.relu(hn * g2[0:1, :] + b2[0:1, :])
    h2r = jax.nn.relu(hn * g2[1:2, :] + b2[1:2, :])
    return jnp.concatenate([h2n, h2r], axis=1)


def _dense_first_body(x_ref, w0_ref, g_ref, b_ref, rt_ref, bi_ref,
                      h2cat_ref, hbase_ref):
    h = jnp.dot(x_ref[...], w0_ref[...], preferred_element_type=jnp.float32)
    h2cat = _ln_relu_cat(h, g_ref[...], b_ref[...])
    h2cat_ref[...] = h2cat
    hbase_ref[...] = h + jnp.dot(h2cat, rt_ref[...],
                                 preferred_element_type=jnp.float32) + bi_ref[...]


def _dense_mid_body(hb_ref, parts_ref, g_ref, b_ref, rt_ref, bi_ref,
                    h2cat_ref, hbase_ref):
    h = hb_ref[...] + parts_ref[0] + parts_ref[1]
    h2cat = _ln_relu_cat(h, g_ref[...], b_ref[...])
    h2cat_ref[...] = h2cat
    hbase_ref[...] = h + jnp.dot(h2cat, rt_ref[...],
                                 preferred_element_type=jnp.float32) + bi_ref[...]


def _final_body(hb_ref, parts_ref, out_ref):
    out_ref[...] = hb_ref[...] + parts_ref[0] + parts_ref[1]


def _edge_body(xj_ref, ea_ref, ear_ref, w1_ref, r2_ref, s2_ref, eb_ref, m_ref):
    xj = xj_ref[...]
    ea_cat = jnp.concatenate([ea_ref[...], ear_ref[...]], axis=1)
    tmp = jnp.dot(xj, w1_ref[...], preferred_element_type=jnp.float32)
    arep = jnp.dot(ea_cat, r2_ref[...], preferred_element_type=jnp.float32)
    p = tmp * arep
    m_ref[...] = (jnp.dot(p, s2_ref[...], preferred_element_type=jnp.float32)
                  + jnp.dot(xj, eb_ref[...], preferred_element_type=jnp.float32))


def _dense_first(x, w0, g2, b2, rtcat, bicat, interpret=False):
    return pl.pallas_call(
        _dense_first_body,
        out_shape=(jax.ShapeDtypeStruct((N, 2 * H), jnp.float32),
                   jax.ShapeDtypeStruct((N, H), jnp.float32)),
        interpret=interpret,
    )(x, w0, g2, b2, rtcat, bicat)


def _dense_mid(hb, parts, g2, b2, rtcat, bicat, interpret=False):
    return pl.pallas_call(
        _dense_mid_body,
        out_shape=(jax.ShapeDtypeStruct((N, 2 * H), jnp.float32),
                   jax.ShapeDtypeStruct((N, H), jnp.float32)),
        interpret=interpret,
    )(hb, parts, g2, b2, rtcat, bicat)


def _final(hb, parts, interpret=False):
    return pl.pallas_call(
        _final_body,
        out_shape=jax.ShapeDtypeStruct((N, H), jnp.float32),
        interpret=interpret,
    )(hb, parts)


def _edge_compute(xj, ea, ea_rev, w1cat, r2, s2cat, ebcat, interpret=False):
    return pl.pallas_call(
        _edge_body,
        grid=(NB,),
        in_specs=[
            pl.BlockSpec((BE, 2 * H), lambda i: (i, 0)),
            pl.BlockSpec((BE, T_OP), lambda i: (i, 0)),
            pl.BlockSpec((BE, T_OP), lambda i: (i, 0)),
            pl.BlockSpec((2 * H, 2 * T_OP * OUT), lambda i: (0, 0)),
            pl.BlockSpec((2 * T_OP, 2 * T_OP * OUT), lambda i: (0, 0)),
            pl.BlockSpec((2 * T_OP * OUT, 2 * OUT), lambda i: (0, 0)),
            pl.BlockSpec((2 * H, 2 * OUT), lambda i: (0, 0)),
        ],
        out_specs=pl.BlockSpec((BE, 2 * OUT), lambda i: (i, 0)),
        out_shape=jax.ShapeDtypeStruct((E, 2 * OUT), jnp.float32),
        interpret=interpret,
    )(xj, ea, ea_rev, w1cat, r2, s2cat, ebcat)


def _gather_nodes(h2cat, src2d):
    return _build_gather_sc(2 * H)(h2cat, src2d)


def _scatter_messages(m, dst2d, zeros_acc):
    return _build_scatter_sc()(m, dst2d, zeros_acc)


def _prep_weights(edge_W, edge_b, root, bias_p, ln_gamma, ln_beta):
    def w1(eW):
        return eW.reshape(T_OP, H, OUT).transpose(1, 2, 0).reshape(H, OUT * T_OP)

    eye_t = jnp.eye(T_OP, dtype=jnp.float32)
    r_b = jnp.tile(eye_t, (1, OUT))
    s_b = jnp.kron(jnp.eye(OUT, dtype=jnp.float32),
                   jnp.ones((T_OP, 1), jnp.float32))
    z = jnp.zeros((T_OP, OUT * T_OP), jnp.float32)
    r2 = jnp.block([[r_b, z], [z, r_b]])
    zs = jnp.zeros((OUT * T_OP, OUT), jnp.float32)
    s2 = jnp.block([[s_b, zs], [zs, s_b]])

    per_layer = []
    for l in range(L):
        w1n, w1r = w1(edge_W[l, 0]), w1(edge_W[l, 1])
        zw = jnp.zeros((H, OUT * T_OP), jnp.float32)
        w1cat = jnp.block([[w1n, zw], [zw, w1r]])
        ebn = edge_b[l, 0].reshape(H, OUT)
        ebr = edge_b[l, 1].reshape(H, OUT)
        zb = jnp.zeros((H, OUT), jnp.float32)
        ebcat = jnp.block([[ebn, zb], [zb, ebr]])
        zr = jnp.zeros((H, OUT), jnp.float32)
        rtcat = jnp.block([[root[l, 0], zr], [zr, root[l, 1]]])
        bicat = jnp.concatenate([bias_p[l, 0], bias_p[l, 1]]).reshape(1, 2 * OUT)
        g2 = ln_gamma[l]
        b2 = ln_beta[l]
        per_layer.append((w1cat, ebcat, rtcat, bicat, g2, b2))
    return r2, s2, per_layer


def _encoder(x, edge_index, edge_attr, W0, ln_gamma, ln_beta, edge_W, edge_b,
             root, bias_p, interpret=False):
    r2, s2, per_layer = _prep_weights(edge_W, edge_b, root, bias_p,
                                      ln_gamma, ln_beta)

    src2d = jnp.pad(edge_index[0].reshape(ROWS, CHUNK),
                    ((0, IDXR - ROWS), (0, 0)))
    dst2d = jnp.pad(edge_index[1].reshape(ROWS, CHUNK),
                    ((0, IDXR - ROWS), (0, 0)))
    zeros_acc = jnp.zeros((N, 2 * OUT), jnp.float32)
    rev2d = jnp.pad(
        ((E - 1) - jnp.arange(E, dtype=jnp.int32)).reshape(ROWS, CHUNK),
        ((0, IDXR - ROWS), (0, 0)))
    if interpret:
        ea_rev = edge_attr[::-1]
    else:
        ea_rev = _build_gather_sc(T_OP)(edge_attr, rev2d)

    hb = None
    parts = None
    for l in range(L):
        w1cat, ebcat, rtcat, bicat, g2, b2 = per_layer[l]
        if l == 0:
            h2cat, hb = _dense_first(x, W0, g2, b2, rtcat, bicat,
                                     interpret=interpret)
        else:
            h2cat, hb = _dense_mid(hb, parts, g2, b2, rtcat, bicat,
                                   interpret=interpret)
        if interpret:
            xj = jnp.take(h2cat, src2d[:ROWS].reshape(-1), axis=0)
        else:
            xj = _gather_nodes(h2cat, src2d)
        m = _edge_compute(xj, edge_attr, ea_rev, w1cat, r2, s2, ebcat,
                          interpret=interpret)
        if interpret:
            aggr = jnp.zeros((N, 2 * OUT), jnp.float32).at[
                dst2d[:ROWS].reshape(-1)].add(m)
            parts = jnp.stack([aggr, jnp.zeros_like(aggr)])
        else:
            parts = _scatter_messages(m, dst2d, zeros_acc)
    return _final(hb, parts, interpret=interpret)


def kernel(x, edge_index, edge_attr, W0, ln_gamma, ln_beta, edge_W, edge_b,
           root, bias_p):
    return _encoder(x, edge_index, edge_attr, W0, ln_gamma, ln_beta,
                    edge_W, edge_b, root, bias_p)


# speedup vs baseline: 1.3702x; 1.0145x over previous
import functools

import jax
import jax.numpy as jnp
from jax import lax
from jax.experimental import pallas as pl
from jax.experimental.pallas import tpu as pltpu
from jax.experimental.pallas import tpu_sc as plsc

N = 10000
E = 160000
T_NODE = 128
T_OP = 16
H = 32
OUT = 16
L = 3

BE = 6400
NB = E // BE
LN_EPS = 1e-5

NC = 2
NS = 16
NW = NC * NS
CHUNK = 128
ROWS = E // CHUNK
IDXR = 1280
RW = IDXR // NW
K_SUP = 13
NPR = N // NS

def _worker_base(c, s):
    wid = s * NC + c
    return wid, wid * 39 + jnp.minimum(wid, 2)


@functools.lru_cache(maxsize=None)
def _build_gather_sc(width):
    @functools.partial(
        pl.kernel,
        out_type=jax.ShapeDtypeStruct((E, width), jnp.float32),
        mesh=plsc.VectorSubcoreMesh(core_axis_name="c", subcore_axis_name="s",
                                    num_cores=NC, num_subcores=NS),
        scratch_types=[
            pltpu.VMEM((RW, CHUNK), jnp.int32),
            pltpu.VMEM((K_SUP * CHUNK, width), jnp.float32),
            pltpu.VMEM((CHUNK, width), jnp.float32),
            pltpu.SemaphoreType.DMA,
        ],
        compiler_params=pltpu.CompilerParams(use_tc_tiling_on_sc=False),
    )
    def _gather_sc(table_hbm, src_hbm, out_hbm, idx_v, buf_v, tail_v, sem):
        c = lax.axis_index("c")
        s = lax.axis_index("s")
        wid, row_base = _worker_base(c, s)
        pltpu.sync_copy(src_hbm.at[pl.ds(row_base, RW)], idx_v)

        for sup in range(3):
            cds = []
            for k in range(K_SUP):
                cds.append(pltpu.async_copy(
                    table_hbm.at[idx_v.at[sup * K_SUP + k]],
                    buf_v.at[pl.ds(k * CHUNK, CHUNK)], sem))
            for cd in cds:
                cd.wait()
            pltpu.sync_copy(
                buf_v,
                out_hbm.at[pl.ds((row_base + sup * K_SUP) * CHUNK,
                                 K_SUP * CHUNK)])

        @pl.when(wid < 2)
        def _tail():
            pltpu.async_copy(table_hbm.at[idx_v.at[39]], tail_v, sem).wait()
            pltpu.sync_copy(tail_v,
                            out_hbm.at[pl.ds((row_base + 39) * CHUNK, CHUNK)])

    return _gather_sc


@functools.lru_cache(maxsize=None)
def _build_scatter_sc():
    @functools.partial(
        pl.kernel,
        out_type=jax.ShapeDtypeStruct((NC, N, 2 * OUT), jnp.float32),
        mesh=plsc.VectorSubcoreMesh(core_axis_name="c", subcore_axis_name="s",
                                    num_cores=NC, num_subcores=NS),
        scratch_types=[
            pltpu.VMEM_SHARED((N, 2 * OUT), jnp.float32),
            pltpu.VMEM((RW, CHUNK), jnp.int32),
            pltpu.VMEM((K_SUP * CHUNK, 2 * OUT), jnp.float32),
            pltpu.SemaphoreType.DMA,
        ],
        compiler_params=pltpu.CompilerParams(use_tc_tiling_on_sc=False),
    )
    def _scatter_sc(m_hbm, dst_hbm, zeros_hbm, out_hbm, acc_sh, idx_v,
                    buf_v, sem):
        c = lax.axis_index("c")
        s = lax.axis_index("s")
        wid, row_base = _worker_base(c, s)

        @pl.when(s == 0)
        def _init():
            pltpu.sync_copy(zeros_hbm, acc_sh)

        pltpu.sync_copy(dst_hbm.at[pl.ds(row_base, RW)], idx_v)
        plsc.subcore_barrier()

        for sup in range(3):
            cds = []
            for k in range(K_SUP):
                cds.append(pltpu.async_copy(
                    m_hbm.at[pl.ds((row_base + sup * K_SUP + k) * CHUNK,
                                   CHUNK)],
                    buf_v.at[pl.ds(k * CHUNK, CHUNK)], sem))
            for cd in cds:
                cd.wait()
            for k in range(K_SUP):
                pltpu.sync_copy(buf_v.at[pl.ds(k * CHUNK, CHUNK)],
                                acc_sh.at[idx_v.at[sup * K_SUP + k]], add=True)

        @pl.when(wid < 2)
        def _tail():
            pltpu.async_copy(
                m_hbm.at[pl.ds((row_base + 39) * CHUNK, CHUNK)],
                buf_v.at[pl.ds(0, CHUNK)], sem).wait()
            pltpu.sync_copy(buf_v.at[pl.ds(0, CHUNK)],
                            acc_sh.at[idx_v.at[39]], add=True)

        plsc.subcore_barrier()
        pltpu.sync_copy(acc_sh.at[pl.ds(s * NPR, NPR)],
                        out_hbm.at[c, pl.ds(s * NPR, NPR)])

    return _scatter_sc


def _ln_relu_cat(h, g2, b2):
    mu = jnp.mean(h, axis=-1, keepdims=True)
    d = h - mu
    var = jnp.mean(d * d, axis=-1, keepdims=True)
    hn = d * lax.rsqrt(var + LN_EPS)
    h2n = jax.nn.relu(hn * g2[0:1, :] + b2[0:1, :])
    h2r = jax.nn.relu(hn * g2[1:2, :] + b2[1:2, :])
    return jnp.concatenate([h2n, h2r], axis=1)


def _dense_first_body(x_ref, w0_ref, g_ref, b_ref, rt_ref, bi_ref,
                      h2cat_ref, hbase_ref):
    h = jnp.dot(x_ref[...], w0_ref[...], preferred_element_type=jnp.float32)
    h2cat = _ln_relu_cat(h, g_ref[...], b_ref[...])
    h2cat_ref[...] = h2cat
    hbase_ref[...] = h + jnp.dot(h2cat, rt_ref[...],
                                 preferred_element_type=jnp.float32) + bi_ref[...]


def _dense_mid_body(hb_ref, parts_ref, g_ref, b_ref, rt_ref, bi_ref,
                    h2cat_ref, hbase_ref):
    h = hb_ref[...] + parts_ref[0] + parts_ref[1]
    h2cat = _ln_relu_cat(h, g_ref[...], b_ref[...])
    h2cat_ref[...] = h2cat
    hbase_ref[...] = h + jnp.dot(h2cat, rt_ref[...],
                                 preferred_element_type=jnp.float32) + bi_ref[...]


def _final_body(hb_ref, parts_ref, out_ref):
    out_ref[...] = hb_ref[...] + parts_ref[0] + parts_ref[1]


def _edge_body(xj_ref, ea_ref, ear_ref, w1_ref, r2_ref, s2_ref, eb_ref, m_ref):
    xj = xj_ref[...]
    ea_cat = jnp.concatenate([ea_ref[...], ear_ref[...]], axis=1)
    tmp = jnp.dot(xj, w1_ref[...], preferred_element_type=jnp.float32)
    arep = jnp.dot(ea_cat, r2_ref[...], preferred_element_type=jnp.float32)
    p = tmp * arep
    m_ref[...] = (jnp.dot(p, s2_ref[...], preferred_element_type=jnp.float32)
                  + jnp.dot(xj, eb_ref[...], preferred_element_type=jnp.float32))


def _dense_first(x, w0, g2, b2, rtcat, bicat, interpret=False):
    return pl.pallas_call(
        _dense_first_body,
        out_shape=(jax.ShapeDtypeStruct((N, 2 * H), jnp.float32),
                   jax.ShapeDtypeStruct((N, H), jnp.float32)),
        interpret=interpret,
    )(x, w0, g2, b2, rtcat, bicat)


def _dense_mid(hb, parts, g2, b2, rtcat, bicat, interpret=False):
    return pl.pallas_call(
        _dense_mid_body,
        out_shape=(jax.ShapeDtypeStruct((N, 2 * H), jnp.float32),
                   jax.ShapeDtypeStruct((N, H), jnp.float32)),
        interpret=interpret,
    )(hb, parts, g2, b2, rtcat, bicat)


def _final(hb, parts, interpret=False):
    return pl.pallas_call(
        _final_body,
        out_shape=jax.ShapeDtypeStruct((N, H), jnp.float32),
        interpret=interpret,
    )(hb, parts)


def _edge_compute(xj, ea, ea_rev, w1cat, r2, s2cat, ebcat, interpret=False):
    return pl.pallas_call(
        _edge_body,
        grid=(NB,),
        in_specs=[
            pl.BlockSpec((BE, 2 * H), lambda i: (i, 0)),
            pl.BlockSpec((BE, T_OP), lambda i: (i, 0)),
            pl.BlockSpec((BE, T_OP), lambda i: (i, 0)),
            pl.BlockSpec((2 * H, 2 * T_OP * OUT), lambda i: (0, 0)),
            pl.BlockSpec((2 * T_OP, 2 * T_OP * OUT), lambda i: (0, 0)),
            pl.BlockSpec((2 * T_OP * OUT, 2 * OUT), lambda i: (0, 0)),
            pl.BlockSpec((2 * H, 2 * OUT), lambda i: (0, 0)),
        ],
        out_specs=pl.BlockSpec((BE, 2 * OUT), lambda i: (i, 0)),
        out_shape=jax.ShapeDtypeStruct((E, 2 * OUT), jnp.float32),
        interpret=interpret,
    )(xj, ea, ea_rev, w1cat, r2, s2cat, ebcat)


def _gather_nodes(h2cat, src2d):
    return _build_gather_sc(2 * H)(h2cat, src2d)


def _scatter_messages(m, dst2d, zeros_acc):
    return _build_scatter_sc()(m, dst2d, zeros_acc)


def _prep_weights(edge_W, edge_b, root, bias_p, ln_gamma, ln_beta):
    def w1(eW):
        return eW.reshape(T_OP, H, OUT).transpose(1, 2, 0).reshape(H, OUT * T_OP)

    eye_t = jnp.eye(T_OP, dtype=jnp.float32)
    r_b = jnp.tile(eye_t, (1, OUT))
    s_b = jnp.kron(jnp.eye(OUT, dtype=jnp.float32),
                   jnp.ones((T_OP, 1), jnp.float32))
    z = jnp.zeros((T_OP, OUT * T_OP), jnp.float32)
    r2 = jnp.block([[r_b, z], [z, r_b]])
    zs = jnp.zeros((OUT * T_OP, OUT), jnp.float32)
    s2 = jnp.block([[s_b, zs], [zs, s_b]])

    per_layer = []
    for l in range(L):
        w1n, w1r = w1(edge_W[l, 0]), w1(edge_W[l, 1])
        zw = jnp.zeros((H, OUT * T_OP), jnp.float32)
        w1cat = jnp.block([[w1n, zw], [zw, w1r]])
        ebn = edge_b[l, 0].reshape(H, OUT)
        ebr = edge_b[l, 1].reshape(H, OUT)
        zb = jnp.zeros((H, OUT), jnp.float32)
        ebcat = jnp.block([[ebn, zb], [zb, ebr]])
        zr = jnp.zeros((H, OUT), jnp.float32)
        rtcat = jnp.block([[root[l, 0], zr], [zr, root[l, 1]]])
        bicat = jnp.concatenate([bias_p[l, 0], bias_p[l, 1]]).reshape(1, 2 * OUT)
        g2 = ln_gamma[l]
        b2 = ln_beta[l]
        per_layer.append((w1cat, ebcat, rtcat, bicat, g2, b2))
    return r2, s2, per_layer


def _encoder(x, edge_index, edge_attr, W0, ln_gamma, ln_beta, edge_W, edge_b,
             root, bias_p, interpret=False):
    r2, s2, per_layer = _prep_weights(edge_W, edge_b, root, bias_p,
                                      ln_gamma, ln_beta)

    src2d = jnp.pad(edge_index[0].reshape(ROWS, CHUNK),
                    ((0, IDXR - ROWS), (0, 0)))
    dst2d = jnp.pad(edge_index[1].reshape(ROWS, CHUNK),
                    ((0, IDXR - ROWS), (0, 0)))
    zeros_acc = jnp.zeros((N, 2 * OUT), jnp.float32)
    rev2d = jnp.pad(
        ((E - 1) - jnp.arange(E, dtype=jnp.int32)).reshape(ROWS, CHUNK),
        ((0, IDXR - ROWS), (0, 0)))
    if interpret:
        ea_rev = edge_attr[::-1]
    else:
        ea_rev = _build_gather_sc(T_OP)(edge_attr, rev2d)

    hb = None
    parts = None
    for l in range(L):
        w1cat, ebcat, rtcat, bicat, g2, b2 = per_layer[l]
        if l == 0:
            h2cat, hb = _dense_first(x, W0, g2, b2, rtcat, bicat,
                                     interpret=interpret)
        else:
            h2cat, hb = _dense_mid(hb, parts, g2, b2, rtcat, bicat,
                                   interpret=interpret)
        if interpret:
            xj = jnp.take(h2cat, src2d[:ROWS].reshape(-1), axis=0)
        else:
            xj = _gather_nodes(h2cat, src2d)
        m = _edge_compute(xj, edge_attr, ea_rev, w1cat, r2, s2, ebcat,
                          interpret=interpret)
        if interpret:
            aggr = jnp.zeros((N, 2 * OUT), jnp.float32).at[
                dst2d[:ROWS].reshape(-1)].add(m)
            parts = jnp.stack([aggr, jnp.zeros_like(aggr)])
        else:
            parts = _scatter_messages(m, dst2d, zeros_acc)
    return _final(hb, parts, interpret=interpret)


def kernel(x, edge_index, edge_attr, W0, ln_gamma, ln_beta, edge_W, edge_b,
           root, bias_p):
    return _encoder(x, edge_index, edge_attr, W0, ln_gamma, ln_beta,
                    edge_W, edge_b, root, bias_p)
